# baseline (speedup 1.0000x reference)
import jax
import jax.numpy as jnp
from jax.experimental import pallas as pl

NUM_TOKENS = 32768
DIM = 768
NUM_EXPERTS = 8
TOP_K = 2

BLOCK_TOKENS = 8192


def _router_kernel(x_ref, w_ref, weights_ref, indices_ref, aux_ref,
                   counts_ref, psum_ref):
    step = pl.program_id(0)
    nsteps = pl.num_programs(0)

    @pl.when(step == 0)
    def _init():
        counts_ref[...] = jnp.zeros_like(counts_ref)
        psum_ref[...] = jnp.zeros_like(psum_ref)

    x = x_ref[...]
    w = w_ref[...]
    logits = jax.lax.dot_general(
        w, x, (((1,), (1,)), ((), ())), preferred_element_type=jnp.float32)

    idx = jax.lax.broadcasted_iota(jnp.int32, logits.shape, 0)
    m1 = jnp.max(logits, axis=0, keepdims=True)
    i1 = jnp.min(jnp.where(logits == m1, idx, NUM_EXPERTS), axis=0,
                 keepdims=True)
    masked = jnp.where(idx == i1, -jnp.inf, logits)
    m2 = jnp.max(masked, axis=0, keepdims=True)
    i2 = jnp.min(jnp.where(masked == m2, idx, NUM_EXPERTS), axis=0,
                 keepdims=True)

    w1 = 1.0 / (1.0 + jnp.exp(m2 - m1))
    w2 = 1.0 - w1
    weights_ref[...] = jnp.concatenate([w1, w2], axis=0)
    indices_ref[...] = jnp.concatenate([i1, i2], axis=0)

    e = jnp.exp(logits - m1)
    probs = e / jnp.sum(e, axis=0, keepdims=True)
    psum_ref[...] += jnp.sum(probs, axis=1, keepdims=True)

    one_hot = ((idx == i1) | (idx == i2)).astype(jnp.float32)
    counts_ref[...] += jnp.sum(one_hot, axis=1, keepdims=True)

    @pl.when(step == nsteps - 1)
    def _finish():
        f = counts_ref[...] / (NUM_TOKENS * TOP_K)
        p = psum_ref[...] / NUM_TOKENS
        aux_ref[...] = NUM_EXPERTS * jnp.sum(f * p, keepdims=True)


@jax.jit
def kernel(x, W):
    grid = NUM_TOKENS // BLOCK_TOKENS
    weights_t, indices_t, aux, counts, _ = pl.pallas_call(
        _router_kernel,
        grid=(grid,),
        in_specs=[
            pl.BlockSpec((BLOCK_TOKENS, DIM), lambda i: (i, 0)),
            pl.BlockSpec((NUM_EXPERTS, DIM), lambda i: (0, 0)),
        ],
        out_specs=[
            pl.BlockSpec((TOP_K, BLOCK_TOKENS), lambda i: (0, i)),
            pl.BlockSpec((TOP_K, BLOCK_TOKENS), lambda i: (0, i)),
            pl.BlockSpec((1, 1), lambda i: (0, 0)),
            pl.BlockSpec((NUM_EXPERTS, 1), lambda i: (0, 0)),
            pl.BlockSpec((NUM_EXPERTS, 1), lambda i: (0, 0)),
        ],
        out_shape=[
            jax.ShapeDtypeStruct((TOP_K, NUM_TOKENS), jnp.float32),
            jax.ShapeDtypeStruct((TOP_K, NUM_TOKENS), jnp.int32),
            jax.ShapeDtypeStruct((1, 1), jnp.float32),
            jax.ShapeDtypeStruct((NUM_EXPERTS, 1), jnp.float32),
            jax.ShapeDtypeStruct((NUM_EXPERTS, 1), jnp.float32),
        ],
    )(x, W)
    return weights_t.T, indices_t.T, aux[0, 0], counts[:, 0]


# speedup vs baseline: 1.0686x; 1.0686x over previous
import jax
import jax.numpy as jnp
from jax.experimental import pallas as pl

NUM_TOKENS = 32768
DIM = 768
NUM_EXPERTS = 8
TOP_K = 2

BLOCK_TOKENS = 4096


def _router_kernel(x_ref, w_ref, weights_ref, indices_ref, aux_ref,
                   counts_ref, psum_ref):
    step = pl.program_id(0)
    nsteps = pl.num_programs(0)

    @pl.when(step == 0)
    def _init():
        counts_ref[...] = jnp.zeros_like(counts_ref)
        psum_ref[...] = jnp.zeros_like(psum_ref)

    x = x_ref[...]
    w = w_ref[...]
    logits = jax.lax.dot_general(
        w, x, (((1,), (1,)), ((), ())), preferred_element_type=jnp.float32)

    idx = jax.lax.broadcasted_iota(jnp.int32, logits.shape, 0)
    m1 = jnp.max(logits, axis=0, keepdims=True)
    i1 = jnp.min(jnp.where(logits == m1, idx, NUM_EXPERTS), axis=0,
                 keepdims=True)
    masked = jnp.where(idx == i1, -jnp.inf, logits)
    m2 = jnp.max(masked, axis=0, keepdims=True)
    i2 = jnp.min(jnp.where(masked == m2, idx, NUM_EXPERTS), axis=0,
                 keepdims=True)

    w1 = 1.0 / (1.0 + jnp.exp(m2 - m1))
    w2 = 1.0 - w1
    weights_ref[...] = jnp.concatenate([w1, w2], axis=0)
    indices_ref[...] = jnp.concatenate([i1, i2], axis=0)

    e = jnp.exp(logits - m1)
    probs = e / jnp.sum(e, axis=0, keepdims=True)
    psum_ref[...] += jnp.sum(probs, axis=1, keepdims=True)

    one_hot = ((idx == i1) | (idx == i2)).astype(jnp.float32)
    counts_ref[...] += jnp.sum(one_hot, axis=1, keepdims=True)

    @pl.when(step == nsteps - 1)
    def _finish():
        f = counts_ref[...] / (NUM_TOKENS * TOP_K)
        p = psum_ref[...] / NUM_TOKENS
        aux_ref[...] = NUM_EXPERTS * jnp.sum(f * p, keepdims=True)


@jax.jit
def kernel(x, W):
    grid = NUM_TOKENS // BLOCK_TOKENS
    weights_t, indices_t, aux, counts, _ = pl.pallas_call(
        _router_kernel,
        grid=(grid,),
        in_specs=[
            pl.BlockSpec((BLOCK_TOKENS, DIM), lambda i: (i, 0)),
            pl.BlockSpec((NUM_EXPERTS, DIM), lambda i: (0, 0)),
        ],
        out_specs=[
            pl.BlockSpec((TOP_K, BLOCK_TOKENS), lambda i: (0, i)),
            pl.BlockSpec((TOP_K, BLOCK_TOKENS), lambda i: (0, i)),
            pl.BlockSpec((1, 1), lambda i: (0, 0)),
            pl.BlockSpec((NUM_EXPERTS, 1), lambda i: (0, 0)),
            pl.BlockSpec((NUM_EXPERTS, 1), lambda i: (0, 0)),
        ],
        out_shape=[
            jax.ShapeDtypeStruct((TOP_K, NUM_TOKENS), jnp.float32),
            jax.ShapeDtypeStruct((TOP_K, NUM_TOKENS), jnp.int32),
            jax.ShapeDtypeStruct((1, 1), jnp.float32),
            jax.ShapeDtypeStruct((NUM_EXPERTS, 1), jnp.float32),
            jax.ShapeDtypeStruct((NUM_EXPERTS, 1), jnp.float32),
        ],
    )(x, W)
    return weights_t.T, indices_t.T, aux[0, 0], counts[:, 0]


# speedup vs baseline: 1.0841x; 1.0145x over previous
import jax
import jax.numpy as jnp
from jax.experimental import pallas as pl

NUM_TOKENS = 32768
DIM = 768
NUM_EXPERTS = 8
TOP_K = 2

BLOCK_TOKENS = 4096


def _router_kernel(xa_ref, xb_ref, w_ref, weights_ref, indices_ref, aux_ref,
                   counts_ref, psum_ref):
    step = pl.program_id(0)
    nsteps = pl.num_programs(0)

    @pl.when(step == 0)
    def _init():
        counts_ref[...] = jnp.zeros_like(counts_ref)
        psum_ref[...] = jnp.zeros_like(psum_ref)

    w = w_ref[...]
    la = jax.lax.dot_general(
        w, xa_ref[...], (((1,), (1,)), ((), ())),
        preferred_element_type=jnp.float32)
    lb = jax.lax.dot_general(
        w, xb_ref[...], (((1,), (1,)), ((), ())),
        preferred_element_type=jnp.float32)
    logits = jnp.concatenate([la, lb], axis=1)

    idx = jax.lax.broadcasted_iota(jnp.int32, logits.shape, 0)
    m1 = jnp.max(logits, axis=0, keepdims=True)
    i1 = jnp.min(jnp.where(logits == m1, idx, NUM_EXPERTS), axis=0,
                 keepdims=True)
    masked = jnp.where(idx == i1, -jnp.inf, logits)
    m2 = jnp.max(masked, axis=0, keepdims=True)
    i2 = jnp.min(jnp.where(masked == m2, idx, NUM_EXPERTS), axis=0,
                 keepdims=True)

    w1 = 1.0 / (1.0 + jnp.exp(m2 - m1))
    w2 = 1.0 - w1
    weights_ref[...] = jnp.concatenate([w1, w2], axis=0)
    indices_ref[...] = jnp.concatenate([i1, i2], axis=0)

    e = jnp.exp(logits - m1)
    probs = e / jnp.sum(e, axis=0, keepdims=True)
    psum_ref[...] += jnp.sum(probs, axis=1, keepdims=True)

    one_hot = ((idx == i1) | (idx == i2)).astype(jnp.float32)
    counts_ref[...] += jnp.sum(one_hot, axis=1, keepdims=True)

    @pl.when(step == nsteps - 1)
    def _finish():
        f = counts_ref[...] / (NUM_TOKENS * TOP_K)
        p = psum_ref[...] / NUM_TOKENS
        aux_ref[...] = NUM_EXPERTS * jnp.sum(f * p, keepdims=True)


@jax.jit
def kernel(x, W):
    grid = NUM_TOKENS // BLOCK_TOKENS
    weights_t, indices_t, aux, counts, _ = pl.pallas_call(
        _router_kernel,
        grid=(grid,),
        in_specs=[
            pl.BlockSpec((BLOCK_TOKENS // 2, DIM), lambda i: (2 * i, 0)),
            pl.BlockSpec((BLOCK_TOKENS // 2, DIM), lambda i: (2 * i + 1, 0)),
            pl.BlockSpec((NUM_EXPERTS, DIM), lambda i: (0, 0)),
        ],
        out_specs=[
            pl.BlockSpec((TOP_K, BLOCK_TOKENS), lambda i: (0, i)),
            pl.BlockSpec((TOP_K, BLOCK_TOKENS), lambda i: (0, i)),
            pl.BlockSpec((1, 1), lambda i: (0, 0)),
            pl.BlockSpec((NUM_EXPERTS, 1), lambda i: (0, 0)),
            pl.BlockSpec((NUM_EXPERTS, 1), lambda i: (0, 0)),
        ],
        out_shape=[
            jax.ShapeDtypeStruct((TOP_K, NUM_TOKENS), jnp.float32),
            jax.ShapeDtypeStruct((TOP_K, NUM_TOKENS), jnp.int32),
            jax.ShapeDtypeStruct((1, 1), jnp.float32),
            jax.ShapeDtypeStruct((NUM_EXPERTS, 1), jnp.float32),
            jax.ShapeDtypeStruct((NUM_EXPERTS, 1), jnp.float32),
        ],
    )(x, x, W)
    return weights_t.T, indices_t.T, aux[0, 0], counts[:, 0]
